# baseline (speedup 1.0000x reference)
import jax
import jax.numpy as jnp
from jax.experimental import pallas as pl
from jax.experimental.pallas import tpu as pltpu

_ROWS = 64
_CHUNK_ROWS = 2
_NBUF = 7
_LAG = 1


def _copy_body(x_hbm, o_hbm, buf, rsem, wsem):
    nch = _ROWS // _CHUNK_ROWS

    def read(i):
        b = i % _NBUF
        return pltpu.make_async_copy(
            x_hbm.at[pl.ds(i * _CHUNK_ROWS, _CHUNK_ROWS)],
            buf.at[b],
            rsem.at[b],
        )

    def write(i):
        b = i % _NBUF
        return pltpu.make_async_copy(
            buf.at[b],
            o_hbm.at[pl.ds(i * _CHUNK_ROWS, _CHUNK_ROWS)],
            wsem.at[b],
        )

    for i in range(nch):
        if i >= _NBUF:
            write(i - _NBUF).wait()
        read(i).start()
        if i >= _LAG:
            j = i - _LAG
            read(j).wait()
            write(j).start()
    for j in range(nch - _LAG, nch):
        read(j).wait()
        write(j).start()
    for j in range(nch - _NBUF, nch):
        write(j).wait()


def kernel(x):
    b, n, f = x.shape
    return pl.pallas_call(
        _copy_body,
        out_shape=jax.ShapeDtypeStruct(x.shape, x.dtype),
        in_specs=[pl.BlockSpec(memory_space=pltpu.MemorySpace.HBM)],
        out_specs=pl.BlockSpec(memory_space=pltpu.MemorySpace.HBM),
        scratch_shapes=[
            pltpu.VMEM((_NBUF, _CHUNK_ROWS, n, f), x.dtype),
            pltpu.SemaphoreType.DMA((_NBUF,)),
            pltpu.SemaphoreType.DMA((_NBUF,)),
        ],
    )(x)


# speedup vs baseline: 1.0024x; 1.0024x over previous
import jax
import jax.numpy as jnp
from jax.experimental import pallas as pl
from jax.experimental.pallas import tpu as pltpu

_CHUNKS = (6, 6, 6, 6, 6, 6, 6, 6, 6, 6, 4)
_SLOT_ROWS = max(_CHUNKS)
_NBUF = 2
_LAG = 1


def _copy_body(x_hbm, o_hbm, buf, rsem, wsem):
    nch = len(_CHUNKS)
    offs = [sum(_CHUNKS[:i]) for i in range(nch)]

    def read(i):
        b = i % _NBUF
        return pltpu.make_async_copy(
            x_hbm.at[pl.ds(offs[i], _CHUNKS[i])],
            buf.at[b, pl.ds(0, _CHUNKS[i])],
            rsem.at[b],
        )

    def write(i):
        b = i % _NBUF
        return pltpu.make_async_copy(
            buf.at[b, pl.ds(0, _CHUNKS[i])],
            o_hbm.at[pl.ds(offs[i], _CHUNKS[i])],
            wsem.at[b],
        )

    for i in range(nch):
        if i >= _NBUF:
            write(i - _NBUF).wait()
        read(i).start()
        if i >= _LAG:
            j = i - _LAG
            read(j).wait()
            write(j).start()
    for j in range(nch - _LAG, nch):
        read(j).wait()
        write(j).start()
    for j in range(nch - _NBUF, nch):
        write(j).wait()


def kernel(x):
    b, n, f = x.shape
    return pl.pallas_call(
        _copy_body,
        out_shape=jax.ShapeDtypeStruct(x.shape, x.dtype),
        in_specs=[pl.BlockSpec(memory_space=pltpu.MemorySpace.HBM)],
        out_specs=pl.BlockSpec(memory_space=pltpu.MemorySpace.HBM),
        scratch_shapes=[
            pltpu.VMEM((_NBUF, _SLOT_ROWS, n, f), x.dtype),
            pltpu.SemaphoreType.DMA((_NBUF,)),
            pltpu.SemaphoreType.DMA((_NBUF,)),
        ],
    )(x)
